# baseline (speedup 1.0000x reference)
import functools

import jax
import jax.numpy as jnp
from jax import lax
from jax.experimental import pallas as pl
from jax.experimental.pallas import tpu as pltpu
from jax.experimental.pallas import tpu_sc as plsc

_NC = 2
_NS = 16
_NW = _NC * _NS

_IVEC = 128


@functools.lru_cache(maxsize=None)
def _make_gather(n: int, d: int, chunk: int):
    per_w = n // _NW
    n_chunks = per_w // chunk
    u = chunk // _IVEC
    assert per_w % chunk == 0 and chunk % _IVEC == 0

    mesh = plsc.VectorSubcoreMesh(core_axis_name="c", subcore_axis_name="s")

    @functools.partial(
        pl.kernel,
        out_type=jax.ShapeDtypeStruct((n, d), jnp.float32),
        mesh=mesh,
        compiler_params=pltpu.CompilerParams(use_tc_tiling_on_sc=False),
        scratch_types=[
            pltpu.VMEM((4, u, _IVEC), jnp.int32),
            pltpu.VMEM((2, chunk, d), jnp.float32),
            pltpu.SemaphoreType.DMA((4,)),
            pltpu.SemaphoreType.DMA((2,)),
            pltpu.SemaphoreType.DMA((2,)),
        ],
    )
    def gather_kernel(idx_hbm, table_hbm, out_hbm, idx_v, rows_v,
                      idx_sem, g_sem, o_sem):
        wid = lax.axis_index("s") * _NC + lax.axis_index("c")
        row0 = wid * (per_w // _IVEC)
        base = wid * per_w

        def start_idx_copy(g, s):
            pltpu.async_copy(
                idx_hbm.at[pl.ds(row0 + g * u, u)],
                idx_v.at[s],
                idx_sem.at[s],
            )

        for g in range(4):
            start_idx_copy(g, g)

        def body(g, _):
            s = lax.rem(g, 4)
            b = lax.rem(g, 2)
            pltpu.make_async_copy(
                idx_hbm.at[pl.ds(0, u)], idx_v.at[s], idx_sem.at[s]
            ).wait()
            @pl.when(g >= 2)
            def _():
                pltpu.make_async_copy(
                    rows_v.at[b], out_hbm.at[pl.ds(0, chunk)], o_sem.at[b]
                ).wait()
            for j in range(u):
                pltpu.async_copy(
                    table_hbm.at[idx_v.at[s, j]],
                    rows_v.at[b, pl.ds(j * _IVEC, _IVEC)],
                    g_sem.at[b],
                )
            for j in range(u):
                pltpu.make_async_copy(
                    table_hbm.at[idx_v.at[s, j]],
                    rows_v.at[b, pl.ds(j * _IVEC, _IVEC)],
                    g_sem.at[b],
                ).wait()
            @pl.when(g + 4 < n_chunks)
            def _():
                start_idx_copy(g + 4, s)
            pltpu.async_copy(
                rows_v.at[b],
                out_hbm.at[pl.ds(base + g * chunk, chunk)],
                o_sem.at[b],
            )
            return _

        lax.fori_loop(0, n_chunks, body, None, unroll=False)

        for b in range(2):
            pltpu.make_async_copy(
                rows_v.at[b], out_hbm.at[pl.ds(0, chunk)], o_sem.at[b]
            ).wait()

    return gather_kernel


def kernel(word_input, weight_all):
    b, l = word_input.shape
    _, d = weight_all.shape
    n = b * l
    idx2d = word_input.reshape(n // _IVEC, _IVEC)
    out = _make_gather(n, d, 640)(idx2d, weight_all)
    return out.reshape(b, l, d)


# speedup vs baseline: 1.0000x; 1.0000x over previous
import functools

import jax
import jax.numpy as jnp
from jax import lax
from jax.experimental import pallas as pl
from jax.experimental.pallas import tpu as pltpu
from jax.experimental.pallas import tpu_sc as plsc

_NC = 2
_NS = 16
_NW = _NC * _NS

_IVEC = 128


@functools.lru_cache(maxsize=None)
def _make_gather(n: int, d: int, chunk: int):
    per_w = n // _NW
    n_chunks = per_w // chunk
    u = chunk // _IVEC
    assert per_w % chunk == 0 and chunk % _IVEC == 0

    mesh = plsc.VectorSubcoreMesh(core_axis_name="c", subcore_axis_name="s")

    @functools.partial(
        pl.kernel,
        out_type=jax.ShapeDtypeStruct((n, d), jnp.float32),
        mesh=mesh,
        compiler_params=pltpu.CompilerParams(use_tc_tiling_on_sc=False),
        scratch_types=[
            pltpu.VMEM((4, u, _IVEC), jnp.int32),
            pltpu.VMEM((3, chunk, d), jnp.float32),
            pltpu.SemaphoreType.DMA((4,)),
            pltpu.SemaphoreType.DMA((3,)),
            pltpu.SemaphoreType.DMA((3,)),
        ],
    )
    def gather_kernel(idx_hbm, table_hbm, out_hbm, idx_v, rows_v,
                      idx_sem, g_sem, o_sem):
        wid = lax.axis_index("s") * _NC + lax.axis_index("c")
        row0 = wid * (per_w // _IVEC)
        base = wid * per_w

        def start_idx_copy(g, s):
            pltpu.async_copy(
                idx_hbm.at[pl.ds(row0 + g * u, u)],
                idx_v.at[s],
                idx_sem.at[s],
            )

        def fire_gathers(g, s, b):
            for j in range(u):
                pltpu.async_copy(
                    table_hbm.at[idx_v.at[s, j]],
                    rows_v.at[b, pl.ds(j * _IVEC, _IVEC)],
                    g_sem.at[b],
                )

        def drain_gathers(b):
            pltpu.make_async_copy(
                rows_v.at[b], out_hbm.at[pl.ds(0, chunk)], g_sem.at[b]
            ).wait()

        def start_out_copy(g, b):
            pltpu.async_copy(
                rows_v.at[b],
                out_hbm.at[pl.ds(base + g * chunk, chunk)],
                o_sem.at[b],
            )

        def wait_out_copy(b):
            pltpu.make_async_copy(
                rows_v.at[b], out_hbm.at[pl.ds(0, chunk)], o_sem.at[b]
            ).wait()

        for g in range(4):
            start_idx_copy(g, g)

        def body(g, _):
            s = lax.rem(g, 4)
            b = lax.rem(g, 3)
            pltpu.make_async_copy(
                idx_hbm.at[pl.ds(0, u)], idx_v.at[s], idx_sem.at[s]
            ).wait()
            @pl.when(g >= 3)
            def _():
                wait_out_copy(b)
            fire_gathers(g, s, b)
            @pl.when(g >= 1)
            def _():
                bp = lax.rem(g + 2, 3)
                sp = lax.rem(g + 3, 4)
                drain_gathers(bp)
                @pl.when(g + 3 < n_chunks)
                def _():
                    start_idx_copy(g + 3, sp)
                start_out_copy(g - 1, bp)
            return _

        lax.fori_loop(0, n_chunks, body, None, unroll=False)

        bl = lax.rem(n_chunks - 1, 3)
        drain_gathers(bl)
        start_out_copy(n_chunks - 1, bl)
        for b in range(3):
            wait_out_copy(b)

    return gather_kernel


def kernel(word_input, weight_all):
    b, l = word_input.shape
    _, d = weight_all.shape
    n = b * l
    idx2d = word_input.reshape(n // _IVEC, _IVEC)
    out = _make_gather(n, d, 640)(idx2d, weight_all)
    return out.reshape(b, l, d)


# speedup vs baseline: 1.2249x; 1.2249x over previous
import functools

import jax
import jax.numpy as jnp
from jax import lax
from jax.experimental import pallas as pl
from jax.experimental.pallas import tpu as pltpu
from jax.experimental.pallas import tpu_sc as plsc

_NC = 2
_NS = 16
_NW = _NC * _NS

_IVEC = 128


@functools.lru_cache(maxsize=None)
def _make_gather(n: int, d: int, chunk: int):
    per_w = n // _NW
    n_chunks = per_w // chunk
    u = chunk // _IVEC
    assert per_w % chunk == 0 and chunk % _IVEC == 0

    mesh = plsc.VectorSubcoreMesh(core_axis_name="c", subcore_axis_name="s")

    @functools.partial(
        pl.kernel,
        out_type=jax.ShapeDtypeStruct((n, d), jnp.float32),
        mesh=mesh,
        scratch_types=[
            pltpu.VMEM((4, chunk), jnp.int32),
            pltpu.VMEM((3, chunk, d), jnp.float32),
            pltpu.SemaphoreType.DMA((4,)),
            pltpu.SemaphoreType.DMA((3,)),
            pltpu.SemaphoreType.DMA((3,)),
        ],
    )
    def gather_kernel(idx_hbm, table_hbm, out_hbm, idx_v, rows_v,
                      idx_sem, g_sem, o_sem):
        wid = lax.axis_index("s") * _NC + lax.axis_index("c")
        base = wid * per_w

        def start_idx_copy(g, s):
            pltpu.async_copy(
                idx_hbm.at[pl.ds(base + g * chunk, chunk)],
                idx_v.at[s],
                idx_sem.at[s],
            )

        def fire_gathers(g, s, b):
            for j in range(u):
                pltpu.async_copy(
                    table_hbm.at[idx_v.at[s, pl.ds(j * _IVEC, _IVEC)]],
                    rows_v.at[b, pl.ds(j * _IVEC, _IVEC)],
                    g_sem.at[b],
                )

        def drain_gathers(b):
            pltpu.make_async_copy(
                rows_v.at[b], out_hbm.at[pl.ds(0, chunk)], g_sem.at[b]
            ).wait()

        def start_out_copy(g, b):
            pltpu.async_copy(
                rows_v.at[b],
                out_hbm.at[pl.ds(base + g * chunk, chunk)],
                o_sem.at[b],
            )

        def wait_out_copy(b):
            pltpu.make_async_copy(
                rows_v.at[b], out_hbm.at[pl.ds(0, chunk)], o_sem.at[b]
            ).wait()

        for g in range(4):
            start_idx_copy(g, g)

        def body(g, _):
            s = lax.rem(g, 4)
            b = lax.rem(g, 3)
            pltpu.make_async_copy(
                idx_hbm.at[pl.ds(0, chunk)], idx_v.at[s], idx_sem.at[s]
            ).wait()
            @pl.when(g >= 3)
            def _():
                wait_out_copy(b)
            fire_gathers(g, s, b)
            @pl.when(g >= 1)
            def _():
                bp = lax.rem(g + 2, 3)
                sp = lax.rem(g + 3, 4)
                drain_gathers(bp)
                @pl.when(g + 3 < n_chunks)
                def _():
                    start_idx_copy(g + 3, sp)
                start_out_copy(g - 1, bp)
            return _

        lax.fori_loop(0, n_chunks, body, None, unroll=False)

        bl = lax.rem(n_chunks - 1, 3)
        drain_gathers(bl)
        start_out_copy(n_chunks - 1, bl)
        for b in range(3):
            wait_out_copy(b)

    return gather_kernel


def kernel(word_input, weight_all):
    b, l = word_input.shape
    v, d = weight_all.shape
    n = b * l
    idx = word_input.reshape(n)
    table128 = jnp.pad(weight_all, ((0, 0), (0, 128 - d)))
    out128 = _make_gather(n, 128, 256)(idx, table128)
    return out128[:, :d].reshape(b, l, d)
